# baseline (speedup 1.0000x reference)
import functools
import jax
import jax.numpy as jnp
from jax import lax
from jax.experimental import pallas as pl
from jax.experimental.pallas import tpu as pltpu
from jax.experimental.pallas import tpu_sc as plsc

_N, _D, _H, _E, _TOPK = 4096, 1024, 1024, 8, 2
_NW = 32
_CHUNK = _N // _NW
_NG = _CHUNK // 16
_TB = 512
_NTB = _N // _TB
_BLK = 128
_BLK_SHIFT = 7
_NB = (_N * _TOPK) // _BLK + _E
_P = _NB * _BLK


def _router_body(x_ref, wg_ref, e0_ref, e1_ref, g0_ref, g1_ref,
                 r0_ref, r1_ref, cnt_ref, cnt_acc):
    i = pl.program_id(0)

    @pl.when(i == 0)
    def _():
        cnt_acc[...] = jnp.zeros_like(cnt_acc)

    logits = jnp.dot(x_ref[...], wg_ref[...],
                     preferred_element_type=jnp.float32)
    colsi = jax.lax.broadcasted_iota(jnp.int32, (_TB, _E), 1)
    m0 = jnp.max(logits, axis=1, keepdims=True)
    e0 = jnp.min(jnp.where(logits == m0, colsi, _E), axis=1, keepdims=True)
    oh0 = (colsi == e0).astype(jnp.float32)
    l1 = jnp.where(colsi == e0, -1e30, logits)
    m1 = jnp.max(l1, axis=1, keepdims=True)
    e1 = jnp.min(jnp.where(l1 == m1, colsi, _E), axis=1, keepdims=True)
    oh1 = (colsi == e1).astype(jnp.float32)
    g0 = 1.0 / (1.0 + jnp.exp(m1 - m0))
    g1 = 1.0 - g0
    rows = jax.lax.broadcasted_iota(jnp.int32, (_TB, _TB), 0)
    cols = jax.lax.broadcasted_iota(jnp.int32, (_TB, _TB), 1)
    lt = (cols < rows).astype(jnp.float32)
    cum0 = jnp.dot(lt, oh0, preferred_element_type=jnp.float32)
    cum1 = jnp.dot(lt, oh1, preferred_element_type=jnp.float32)
    cnt = cnt_acc[...]
    tot0 = jnp.sum(oh0, axis=0, keepdims=True)
    tot1 = jnp.sum(oh1, axis=0, keepdims=True)
    r0 = jnp.sum(oh0 * (cnt + cum0), axis=1, keepdims=True)
    r1 = jnp.sum(oh1 * (cnt + tot0 + cum1), axis=1, keepdims=True)
    new_cnt = cnt + tot0 + tot1
    cnt_acc[...] = new_cnt
    cnt_ref[...] = new_cnt
    e0_ref[...] = e0
    e1_ref[...] = e1
    g0_ref[...] = g0
    g1_ref[...] = g1
    r0_ref[...] = r0.astype(jnp.int32)
    r1_ref[...] = r1.astype(jnp.int32)


def _router(x, w_gate):
    col = lambda dt: jax.ShapeDtypeStruct((_N, 1), dt)
    out_shapes = (col(jnp.int32), col(jnp.int32), col(jnp.float32),
                  col(jnp.float32), col(jnp.int32), col(jnp.int32),
                  jax.ShapeDtypeStruct((1, _E), jnp.float32))
    colspec = lambda: pl.BlockSpec((_TB, 1), lambda i: (i, 0))
    return pl.pallas_call(
        _router_body,
        grid=(_NTB,),
        in_specs=[
            pl.BlockSpec((_TB, _D), lambda i: (i, 0)),
            pl.BlockSpec((_D, _E), lambda i: (0, 0)),
        ],
        out_specs=(colspec(), colspec(), colspec(), colspec(),
                   colspec(), colspec(),
                   pl.BlockSpec((1, _E), lambda i: (0, 0))),
        out_shape=out_shapes,
        scratch_shapes=[pltpu.VMEM((1, _E), jnp.float32)],
    )(x, w_gate)


def _offsets_body(cnt_ref, off_ref, blk_ref):
    c = jnp.round(cnt_ref[...]).astype(jnp.int32)
    nb = (c + (_BLK - 1)) >> _BLK_SHIFT
    cpad = (nb << _BLK_SHIFT).astype(jnp.float32)
    f = jax.lax.broadcasted_iota(jnp.int32, (_E, _E), 0)
    e = jax.lax.broadcasted_iota(jnp.int32, (_E, _E), 1)
    ut = (f < e).astype(jnp.float32)
    off = jnp.dot(cpad, ut, preferred_element_type=jnp.float32)
    off_ref[...] = jnp.round(off).astype(jnp.int32)
    ib = jax.lax.broadcasted_iota(jnp.int32, (_NB, _E), 0) * _BLK
    le = (off.astype(jnp.int32) <= ib).astype(jnp.int32)
    blk = jnp.sum(le, axis=1, keepdims=True) - 1
    blk_ref[...] = jnp.clip(blk, 0, _E - 1)


def _offsets(cnt):
    return pl.pallas_call(
        _offsets_body,
        in_specs=[pl.BlockSpec((1, _E), lambda: (0, 0))],
        out_specs=(pl.BlockSpec((1, _E), lambda: (0, 0)),
                   pl.BlockSpec((_NB, 1), lambda: (0, 0))),
        out_shape=(jax.ShapeDtypeStruct((1, _E), jnp.int32),
                   jax.ShapeDtypeStruct((_NB, 1), jnp.int32)),
    )(cnt)


def _gmm_body(m_ref, xs_ref, gs_ref, w1_ref, w2_ref, ys_ref):
    h = jnp.maximum(
        jnp.dot(xs_ref[...], w1_ref[0], preferred_element_type=jnp.float32),
        0.0)
    o = jnp.dot(h, w2_ref[0], preferred_element_type=jnp.float32)
    ys_ref[...] = o * gs_ref[...]


def _grouped_matmul(xs, gs, W1, W2, blk_expert):
    grid_spec = pltpu.PrefetchScalarGridSpec(
        num_scalar_prefetch=1,
        grid=(_NB,),
        in_specs=[
            pl.BlockSpec((_BLK, _D), lambda i, m: (i, 0)),
            pl.BlockSpec((_BLK, 1), lambda i, m: (i, 0)),
            pl.BlockSpec((1, _D, _H), lambda i, m: (m[i], 0, 0)),
            pl.BlockSpec((1, _H, _D), lambda i, m: (m[i], 0, 0)),
        ],
        out_specs=pl.BlockSpec((_BLK, _D), lambda i, m: (i, 0)),
    )
    return pl.pallas_call(
        _gmm_body,
        grid_spec=grid_spec,
        out_shape=jax.ShapeDtypeStruct((_P, _D), jnp.float32),
    )(blk_expert, xs, gs.reshape(_P, 1), W1, W2)


def _dispatch_body(x_hbm, e0_hbm, e1_hbm, r0_hbm, r1_hbm, g0_hbm, g1_hbm,
                   off_hbm, xs_hbm, gs_hbm, pos0_hbm, pos1_hbm,
                   e0_v, e1_v, r0_v, r1_v, g0_v, g1_v, off_v,
                   pos0_v, pos1_v, xrow_v,
                   sem_m, sem_r0, sem_r1, sem_r2, sem_r3,
                   sem_w0, sem_w1, sem_w2, sem_w3):
    wid = lax.axis_index("s") * 2 + lax.axis_index("c")
    base = wid * _CHUNK
    sem_r = (sem_r0, sem_r1, sem_r2, sem_r3)
    sem_w = (sem_w0, sem_w1, sem_w2, sem_w3)

    def fire_read(t):
        return pltpu.async_copy(x_hbm.at[pl.ds(base + t * 16, 16)],
                                xrow_v.at[t % 4], sem_r[t % 4])

    reads = {t: fire_read(t) for t in range(4)}
    meta = [
        pltpu.async_copy(e0_hbm.at[pl.ds(base, _CHUNK)], e0_v, sem_m),
        pltpu.async_copy(e1_hbm.at[pl.ds(base, _CHUNK)], e1_v, sem_m),
        pltpu.async_copy(r0_hbm.at[pl.ds(base, _CHUNK)], r0_v, sem_m),
        pltpu.async_copy(r1_hbm.at[pl.ds(base, _CHUNK)], r1_v, sem_m),
        pltpu.async_copy(g0_hbm.at[pl.ds(base, _CHUNK)], g0_v, sem_m),
        pltpu.async_copy(g1_hbm.at[pl.ds(base, _CHUNK)], g1_v, sem_m),
        pltpu.async_copy(off_hbm, off_v, sem_m),
    ]
    for d in meta:
        d.wait()
    for t in range(_NG):
        sl = pl.ds(t * 16, 16)
        pos0_v[sl] = plsc.load_gather(off_v, [e0_v[sl]]) + r0_v[sl]
        pos1_v[sl] = plsc.load_gather(off_v, [e1_v[sl]]) + r1_v[sl]
    tail = [
        pltpu.async_copy(pos0_v, pos0_hbm.at[pl.ds(base, _CHUNK)], sem_m),
        pltpu.async_copy(pos1_v, pos1_hbm.at[pl.ds(base, _CHUNK)], sem_m),
        pltpu.async_copy(g0_v, gs_hbm.at[pos0_v], sem_m),
        pltpu.async_copy(g1_v, gs_hbm.at[pos1_v], sem_m),
    ]

    writes = {}
    for t in range(_NG):
        sl = pl.ds(t * 16, 16)
        reads[t].wait()
        writes[t] = (
            pltpu.async_copy(xrow_v.at[t % 4], xs_hbm.at[pos0_v[sl]],
                             sem_w[t % 4]),
            pltpu.async_copy(xrow_v.at[t % 4], xs_hbm.at[pos1_v[sl]],
                             sem_w[t % 4]),
        )
        if t + 4 < _NG:
            writes[t][0].wait()
            writes[t][1].wait()
            reads[t + 4] = fire_read(t + 4)
    for t in range(_NG - 4, _NG):
        writes[t][0].wait()
        writes[t][1].wait()
    for d in tail:
        d.wait()


def _dispatch(x, e0, e1, r0, r1, g0, g1, off):
    mesh = plsc.VectorSubcoreMesh(core_axis_name="c", subcore_axis_name="s")
    f = pl.kernel(
        _dispatch_body,
        mesh=mesh,
        out_type=(jax.ShapeDtypeStruct((_P, _D), jnp.float32),
                  jax.ShapeDtypeStruct((_P,), jnp.float32),
                  jax.ShapeDtypeStruct((_N,), jnp.int32),
                  jax.ShapeDtypeStruct((_N,), jnp.int32)),
        scratch_types=[
            pltpu.VMEM((_CHUNK,), jnp.int32),
            pltpu.VMEM((_CHUNK,), jnp.int32),
            pltpu.VMEM((_CHUNK,), jnp.int32),
            pltpu.VMEM((_CHUNK,), jnp.int32),
            pltpu.VMEM((_CHUNK,), jnp.float32),
            pltpu.VMEM((_CHUNK,), jnp.float32),
            pltpu.VMEM((_E,), jnp.int32),
            pltpu.VMEM((_CHUNK,), jnp.int32),
            pltpu.VMEM((_CHUNK,), jnp.int32),
            pltpu.VMEM((4, 16, _D), jnp.float32),
            pltpu.SemaphoreType.DMA,
            pltpu.SemaphoreType.DMA,
            pltpu.SemaphoreType.DMA,
            pltpu.SemaphoreType.DMA,
            pltpu.SemaphoreType.DMA,
            pltpu.SemaphoreType.DMA,
            pltpu.SemaphoreType.DMA,
            pltpu.SemaphoreType.DMA,
            pltpu.SemaphoreType.DMA,
        ],
        compiler_params=pltpu.CompilerParams(needs_layout_passes=False),
    )
    return f(x, e0, e1, r0, r1, g0, g1, off)


def _combine_body(ys_hbm, pos0_hbm, pos1_hbm, y_hbm,
                  pos0_v, pos1_v, a_v, b_v, o_v,
                  sem_r0, sem_r1, sem_w0, sem_w1):
    wid = lax.axis_index("s") * 2 + lax.axis_index("c")
    base = wid * _CHUNK
    sem_r = (sem_r0, sem_r1)
    sem_w = (sem_w0, sem_w1)
    pltpu.sync_copy(pos0_hbm.at[pl.ds(base, _CHUNK)], pos0_v)
    pltpu.sync_copy(pos1_hbm.at[pl.ds(base, _CHUNK)], pos1_v)

    def fire(t):
        p = t & 1
        sl = pl.ds(t * 16, 16)
        return (pltpu.async_copy(ys_hbm.at[pos0_v[sl]], a_v.at[p], sem_r[p]),
                pltpu.async_copy(ys_hbm.at[pos1_v[sl]], b_v.at[p], sem_r[p]))

    reads = {0: fire(0), 1: fire(1)}
    writes = {}
    for t in range(_NG):
        p = t & 1
        reads[t][0].wait()
        reads[t][1].wait()
        if t >= 2:
            writes[t - 2].wait()

        def row(j, _):
            def col(c, _):
                csl = pl.ds(c * 16, 16)
                o_v[p, j, csl] = a_v[p, j, csl] + b_v[p, j, csl]
                return 0
            lax.fori_loop(0, _D // 16, col, 0)
            return 0
        lax.fori_loop(0, 16, row, 0)
        writes[t] = pltpu.async_copy(
            o_v.at[p], y_hbm.at[pl.ds(base + t * 16, 16)], sem_w[p])
        if t + 2 < _NG:
            reads[t + 2] = fire(t + 2)
    writes[_NG - 2].wait()
    writes[_NG - 1].wait()


def _combine(ys, pos0, pos1):
    mesh = plsc.VectorSubcoreMesh(core_axis_name="c", subcore_axis_name="s")
    f = pl.kernel(
        _combine_body,
        mesh=mesh,
        out_type=jax.ShapeDtypeStruct((_N, _D), jnp.float32),
        scratch_types=[
            pltpu.VMEM((_CHUNK,), jnp.int32),
            pltpu.VMEM((_CHUNK,), jnp.int32),
            pltpu.VMEM((2, 16, _D), jnp.float32),
            pltpu.VMEM((2, 16, _D), jnp.float32),
            pltpu.VMEM((2, 16, _D), jnp.float32),
            pltpu.SemaphoreType.DMA,
            pltpu.SemaphoreType.DMA,
            pltpu.SemaphoreType.DMA,
            pltpu.SemaphoreType.DMA,
        ],
        compiler_params=pltpu.CompilerParams(needs_layout_passes=False),
    )
    return f(ys, pos0, pos1)


def kernel(x, w_gate, W1, W2):
    e0, e1, g0, g1, r0, r1, cnt = _router(x, w_gate)
    off, blk_expert = _offsets(cnt)
    e0, e1, r0, r1 = (a.reshape(_N) for a in (e0, e1, r0, r1))
    g0, g1 = g0.reshape(_N), g1.reshape(_N)
    xs, gs, pos0, pos1 = _dispatch(x, e0, e1, r0, r1, g0, g1, off.reshape(_E))
    ys = _grouped_matmul(xs, gs, W1, W2, blk_expert.reshape(_NB))
    return _combine(ys, pos0, pos1)


# speedup vs baseline: 1.0815x; 1.0815x over previous
import functools
import jax
import jax.numpy as jnp
from jax import lax
from jax.experimental import pallas as pl
from jax.experimental.pallas import tpu as pltpu
from jax.experimental.pallas import tpu_sc as plsc

_N, _D, _H, _E, _TOPK = 4096, 1024, 1024, 8, 2
_NW = 32
_CHUNK = _N // _NW
_NG = _CHUNK // 16
_TB = 512
_NTB = _N // _TB
_BLK = 256
_BLK_SHIFT = 8
_NB = (_N * _TOPK) // _BLK + _E
_P = _NB * _BLK


def _router_body(x_ref, wg_ref, e0_ref, e1_ref, g0_ref, g1_ref,
                 r0_ref, r1_ref, cnt_ref, cnt_acc):
    i = pl.program_id(0)

    @pl.when(i == 0)
    def _():
        cnt_acc[...] = jnp.zeros_like(cnt_acc)

    logits = jnp.dot(x_ref[...], wg_ref[...],
                     preferred_element_type=jnp.float32)
    colsi = jax.lax.broadcasted_iota(jnp.int32, (_TB, _E), 1)
    m0 = jnp.max(logits, axis=1, keepdims=True)
    e0 = jnp.min(jnp.where(logits == m0, colsi, _E), axis=1, keepdims=True)
    oh0 = (colsi == e0).astype(jnp.float32)
    l1 = jnp.where(colsi == e0, -1e30, logits)
    m1 = jnp.max(l1, axis=1, keepdims=True)
    e1 = jnp.min(jnp.where(l1 == m1, colsi, _E), axis=1, keepdims=True)
    oh1 = (colsi == e1).astype(jnp.float32)
    g0 = 1.0 / (1.0 + jnp.exp(m1 - m0))
    g1 = 1.0 - g0
    rows = jax.lax.broadcasted_iota(jnp.int32, (_TB, _TB), 0)
    cols = jax.lax.broadcasted_iota(jnp.int32, (_TB, _TB), 1)
    lt = (cols < rows).astype(jnp.float32)
    cum0 = jnp.dot(lt, oh0, preferred_element_type=jnp.float32)
    cum1 = jnp.dot(lt, oh1, preferred_element_type=jnp.float32)
    cnt = cnt_acc[...]
    tot0 = jnp.sum(oh0, axis=0, keepdims=True)
    tot1 = jnp.sum(oh1, axis=0, keepdims=True)
    r0 = jnp.sum(oh0 * (cnt + cum0), axis=1, keepdims=True)
    r1 = jnp.sum(oh1 * (cnt + tot0 + cum1), axis=1, keepdims=True)
    new_cnt = cnt + tot0 + tot1
    cnt_acc[...] = new_cnt
    cnt_ref[...] = new_cnt
    e0_ref[...] = e0
    e1_ref[...] = e1
    g0_ref[...] = g0
    g1_ref[...] = g1
    r0_ref[...] = r0.astype(jnp.int32)
    r1_ref[...] = r1.astype(jnp.int32)


def _router(x, w_gate):
    col = lambda dt: jax.ShapeDtypeStruct((_N, 1), dt)
    out_shapes = (col(jnp.int32), col(jnp.int32), col(jnp.float32),
                  col(jnp.float32), col(jnp.int32), col(jnp.int32),
                  jax.ShapeDtypeStruct((1, _E), jnp.float32))
    colspec = lambda: pl.BlockSpec((_TB, 1), lambda i: (i, 0))
    return pl.pallas_call(
        _router_body,
        grid=(_NTB,),
        in_specs=[
            pl.BlockSpec((_TB, _D), lambda i: (i, 0)),
            pl.BlockSpec((_D, _E), lambda i: (0, 0)),
        ],
        out_specs=(colspec(), colspec(), colspec(), colspec(),
                   colspec(), colspec(),
                   pl.BlockSpec((1, _E), lambda i: (0, 0))),
        out_shape=out_shapes,
        scratch_shapes=[pltpu.VMEM((1, _E), jnp.float32)],
    )(x, w_gate)


def _offsets_body(cnt_ref, off_ref, blk_ref):
    c = jnp.round(cnt_ref[...]).astype(jnp.int32)
    nb = (c + (_BLK - 1)) >> _BLK_SHIFT
    cpad = (nb << _BLK_SHIFT).astype(jnp.float32)
    f = jax.lax.broadcasted_iota(jnp.int32, (_E, _E), 0)
    e = jax.lax.broadcasted_iota(jnp.int32, (_E, _E), 1)
    ut = (f < e).astype(jnp.float32)
    off = jnp.dot(cpad, ut, preferred_element_type=jnp.float32)
    off_ref[...] = jnp.round(off).astype(jnp.int32)
    ib = jax.lax.broadcasted_iota(jnp.int32, (_NB, _E), 0) * _BLK
    le = (off.astype(jnp.int32) <= ib).astype(jnp.int32)
    blk = jnp.sum(le, axis=1, keepdims=True) - 1
    blk_ref[...] = jnp.clip(blk, 0, _E - 1)


def _offsets(cnt):
    return pl.pallas_call(
        _offsets_body,
        in_specs=[pl.BlockSpec((1, _E), lambda: (0, 0))],
        out_specs=(pl.BlockSpec((1, _E), lambda: (0, 0)),
                   pl.BlockSpec((_NB, 1), lambda: (0, 0))),
        out_shape=(jax.ShapeDtypeStruct((1, _E), jnp.int32),
                   jax.ShapeDtypeStruct((_NB, 1), jnp.int32)),
    )(cnt)


def _gmm_body(m_ref, xs_ref, gs_ref, w1_ref, w2_ref, ys_ref):
    h = jnp.maximum(
        jnp.dot(xs_ref[...], w1_ref[0], preferred_element_type=jnp.float32),
        0.0)
    o = jnp.dot(h, w2_ref[0], preferred_element_type=jnp.float32)
    ys_ref[...] = o * gs_ref[...]


def _grouped_matmul(xs, gs, W1, W2, blk_expert):
    grid_spec = pltpu.PrefetchScalarGridSpec(
        num_scalar_prefetch=1,
        grid=(_NB,),
        in_specs=[
            pl.BlockSpec((_BLK, _D), lambda i, m: (i, 0)),
            pl.BlockSpec((_BLK, 1), lambda i, m: (i, 0)),
            pl.BlockSpec((1, _D, _H), lambda i, m: (m[i], 0, 0)),
            pl.BlockSpec((1, _H, _D), lambda i, m: (m[i], 0, 0)),
        ],
        out_specs=pl.BlockSpec((_BLK, _D), lambda i, m: (i, 0)),
    )
    return pl.pallas_call(
        _gmm_body,
        grid_spec=grid_spec,
        out_shape=jax.ShapeDtypeStruct((_P, _D), jnp.float32),
    )(blk_expert, xs, gs.reshape(_P, 1), W1, W2)


def _dispatch_body(x_hbm, e0_hbm, e1_hbm, r0_hbm, r1_hbm, g0_hbm, g1_hbm,
                   off_hbm, xs_hbm, gs_hbm, pos0_hbm, pos1_hbm,
                   e0_v, e1_v, r0_v, r1_v, g0_v, g1_v, off_v,
                   pos0_v, pos1_v, xrow_v,
                   sem_m, sem_r0, sem_r1, sem_r2, sem_r3,
                   sem_w0, sem_w1, sem_w2, sem_w3):
    wid = lax.axis_index("s") * 2 + lax.axis_index("c")
    base = wid * _CHUNK
    sem_r = (sem_r0, sem_r1, sem_r2, sem_r3)
    sem_w = (sem_w0, sem_w1, sem_w2, sem_w3)

    def fire_read(t):
        return pltpu.async_copy(x_hbm.at[pl.ds(base + t * 16, 16)],
                                xrow_v.at[t % 4], sem_r[t % 4])

    reads = {t: fire_read(t) for t in range(4)}
    meta = [
        pltpu.async_copy(e0_hbm.at[pl.ds(base, _CHUNK)], e0_v, sem_m),
        pltpu.async_copy(e1_hbm.at[pl.ds(base, _CHUNK)], e1_v, sem_m),
        pltpu.async_copy(r0_hbm.at[pl.ds(base, _CHUNK)], r0_v, sem_m),
        pltpu.async_copy(r1_hbm.at[pl.ds(base, _CHUNK)], r1_v, sem_m),
        pltpu.async_copy(g0_hbm.at[pl.ds(base, _CHUNK)], g0_v, sem_m),
        pltpu.async_copy(g1_hbm.at[pl.ds(base, _CHUNK)], g1_v, sem_m),
        pltpu.async_copy(off_hbm, off_v, sem_m),
    ]
    for d in meta:
        d.wait()
    for t in range(_NG):
        sl = pl.ds(t * 16, 16)
        pos0_v[sl] = plsc.load_gather(off_v, [e0_v[sl]]) + r0_v[sl]
        pos1_v[sl] = plsc.load_gather(off_v, [e1_v[sl]]) + r1_v[sl]
    tail = [
        pltpu.async_copy(pos0_v, pos0_hbm.at[pl.ds(base, _CHUNK)], sem_m),
        pltpu.async_copy(pos1_v, pos1_hbm.at[pl.ds(base, _CHUNK)], sem_m),
        pltpu.async_copy(g0_v, gs_hbm.at[pos0_v], sem_m),
        pltpu.async_copy(g1_v, gs_hbm.at[pos1_v], sem_m),
    ]

    writes = {}
    for t in range(_NG):
        sl = pl.ds(t * 16, 16)
        reads[t].wait()
        writes[t] = (
            pltpu.async_copy(xrow_v.at[t % 4], xs_hbm.at[pos0_v[sl]],
                             sem_w[t % 4]),
            pltpu.async_copy(xrow_v.at[t % 4], xs_hbm.at[pos1_v[sl]],
                             sem_w[t % 4]),
        )
        if t + 4 < _NG:
            writes[t][0].wait()
            writes[t][1].wait()
            reads[t + 4] = fire_read(t + 4)
    for t in range(_NG - 4, _NG):
        writes[t][0].wait()
        writes[t][1].wait()
    for d in tail:
        d.wait()


def _dispatch(x, e0, e1, r0, r1, g0, g1, off):
    mesh = plsc.VectorSubcoreMesh(core_axis_name="c", subcore_axis_name="s")
    f = pl.kernel(
        _dispatch_body,
        mesh=mesh,
        out_type=(jax.ShapeDtypeStruct((_P, _D), jnp.float32),
                  jax.ShapeDtypeStruct((_P,), jnp.float32),
                  jax.ShapeDtypeStruct((_N,), jnp.int32),
                  jax.ShapeDtypeStruct((_N,), jnp.int32)),
        scratch_types=[
            pltpu.VMEM((_CHUNK,), jnp.int32),
            pltpu.VMEM((_CHUNK,), jnp.int32),
            pltpu.VMEM((_CHUNK,), jnp.int32),
            pltpu.VMEM((_CHUNK,), jnp.int32),
            pltpu.VMEM((_CHUNK,), jnp.float32),
            pltpu.VMEM((_CHUNK,), jnp.float32),
            pltpu.VMEM((_E,), jnp.int32),
            pltpu.VMEM((_CHUNK,), jnp.int32),
            pltpu.VMEM((_CHUNK,), jnp.int32),
            pltpu.VMEM((4, 16, _D), jnp.float32),
            pltpu.SemaphoreType.DMA,
            pltpu.SemaphoreType.DMA,
            pltpu.SemaphoreType.DMA,
            pltpu.SemaphoreType.DMA,
            pltpu.SemaphoreType.DMA,
            pltpu.SemaphoreType.DMA,
            pltpu.SemaphoreType.DMA,
            pltpu.SemaphoreType.DMA,
            pltpu.SemaphoreType.DMA,
        ],
        compiler_params=pltpu.CompilerParams(needs_layout_passes=False),
    )
    return f(x, e0, e1, r0, r1, g0, g1, off)


def _combine_body(ys_hbm, pos0_hbm, pos1_hbm, y_hbm,
                  pos0_v, pos1_v, a_v, b_v, o_v,
                  sem_r0, sem_r1, sem_w0, sem_w1):
    wid = lax.axis_index("s") * 2 + lax.axis_index("c")
    base = wid * _CHUNK
    sem_r = (sem_r0, sem_r1)
    sem_w = (sem_w0, sem_w1)
    pltpu.sync_copy(pos0_hbm.at[pl.ds(base, _CHUNK)], pos0_v)
    pltpu.sync_copy(pos1_hbm.at[pl.ds(base, _CHUNK)], pos1_v)

    def fire(t):
        p = t & 1
        sl = pl.ds(t * 16, 16)
        return (pltpu.async_copy(ys_hbm.at[pos0_v[sl]], a_v.at[p], sem_r[p]),
                pltpu.async_copy(ys_hbm.at[pos1_v[sl]], b_v.at[p], sem_r[p]))

    reads = {0: fire(0), 1: fire(1)}
    writes = {}
    for t in range(_NG):
        p = t & 1
        reads[t][0].wait()
        reads[t][1].wait()
        if t >= 2:
            writes[t - 2].wait()

        def row(j, _):
            def col(c, _):
                csl = pl.ds(c * 16, 16)
                o_v[p, j, csl] = a_v[p, j, csl] + b_v[p, j, csl]
                return 0
            lax.fori_loop(0, _D // 16, col, 0)
            return 0
        lax.fori_loop(0, 16, row, 0)
        writes[t] = pltpu.async_copy(
            o_v.at[p], y_hbm.at[pl.ds(base + t * 16, 16)], sem_w[p])
        if t + 2 < _NG:
            reads[t + 2] = fire(t + 2)
    writes[_NG - 2].wait()
    writes[_NG - 1].wait()


def _combine(ys, pos0, pos1):
    mesh = plsc.VectorSubcoreMesh(core_axis_name="c", subcore_axis_name="s")
    f = pl.kernel(
        _combine_body,
        mesh=mesh,
        out_type=jax.ShapeDtypeStruct((_N, _D), jnp.float32),
        scratch_types=[
            pltpu.VMEM((_CHUNK,), jnp.int32),
            pltpu.VMEM((_CHUNK,), jnp.int32),
            pltpu.VMEM((2, 16, _D), jnp.float32),
            pltpu.VMEM((2, 16, _D), jnp.float32),
            pltpu.VMEM((2, 16, _D), jnp.float32),
            pltpu.SemaphoreType.DMA,
            pltpu.SemaphoreType.DMA,
            pltpu.SemaphoreType.DMA,
            pltpu.SemaphoreType.DMA,
        ],
        compiler_params=pltpu.CompilerParams(needs_layout_passes=False),
    )
    return f(ys, pos0, pos1)


def kernel(x, w_gate, W1, W2):
    e0, e1, g0, g1, r0, r1, cnt = _router(x, w_gate)
    off, blk_expert = _offsets(cnt)
    e0, e1, r0, r1 = (a.reshape(_N) for a in (e0, e1, r0, r1))
    g0, g1 = g0.reshape(_N), g1.reshape(_N)
    xs, gs, pos0, pos1 = _dispatch(x, e0, e1, r0, r1, g0, g1, off.reshape(_E))
    ys = _grouped_matmul(xs, gs, W1, W2, blk_expert.reshape(_NB))
    return _combine(ys, pos0, pos1)


# speedup vs baseline: 1.1374x; 1.0517x over previous
import functools
import jax
import jax.numpy as jnp
from jax import lax
from jax.experimental import pallas as pl
from jax.experimental.pallas import tpu as pltpu
from jax.experimental.pallas import tpu_sc as plsc

_N, _D, _H, _E, _TOPK = 4096, 1024, 1024, 8, 2
_NW = 32
_CHUNK = _N // _NW
_NG = _CHUNK // 16
_TB = 512
_NTB = _N // _TB
_BLK = 256
_BLK_SHIFT = 8
_NB = (_N * _TOPK) // _BLK + _E
_P = _NB * _BLK


def _router_body(x_ref, wg_ref, e0_ref, e1_ref, g0_ref, g1_ref,
                 r0_ref, r1_ref, cnt_ref, cnt_acc):
    i = pl.program_id(0)

    @pl.when(i == 0)
    def _():
        cnt_acc[...] = jnp.zeros_like(cnt_acc)

    logits = jnp.dot(x_ref[...], wg_ref[...],
                     preferred_element_type=jnp.float32)
    colsi = jax.lax.broadcasted_iota(jnp.int32, (_TB, _E), 1)
    m0 = jnp.max(logits, axis=1, keepdims=True)
    e0 = jnp.min(jnp.where(logits == m0, colsi, _E), axis=1, keepdims=True)
    oh0 = (colsi == e0).astype(jnp.float32)
    l1 = jnp.where(colsi == e0, -1e30, logits)
    m1 = jnp.max(l1, axis=1, keepdims=True)
    e1 = jnp.min(jnp.where(l1 == m1, colsi, _E), axis=1, keepdims=True)
    oh1 = (colsi == e1).astype(jnp.float32)
    g0 = 1.0 / (1.0 + jnp.exp(m1 - m0))
    g1 = 1.0 - g0
    rows = jax.lax.broadcasted_iota(jnp.int32, (_TB, _TB), 0)
    cols = jax.lax.broadcasted_iota(jnp.int32, (_TB, _TB), 1)
    lt = (cols < rows).astype(jnp.float32)
    cum0 = jnp.dot(lt, oh0, preferred_element_type=jnp.float32)
    cum1 = jnp.dot(lt, oh1, preferred_element_type=jnp.float32)
    cnt = cnt_acc[...]
    tot0 = jnp.sum(oh0, axis=0, keepdims=True)
    tot1 = jnp.sum(oh1, axis=0, keepdims=True)
    r0 = jnp.sum(oh0 * (cnt + cum0), axis=1, keepdims=True)
    r1 = jnp.sum(oh1 * (cnt + tot0 + cum1), axis=1, keepdims=True)
    new_cnt = cnt + tot0 + tot1
    cnt_acc[...] = new_cnt
    cnt_ref[...] = new_cnt
    e0_ref[...] = e0
    e1_ref[...] = e1
    g0_ref[...] = g0
    g1_ref[...] = g1
    r0_ref[...] = r0.astype(jnp.int32)
    r1_ref[...] = r1.astype(jnp.int32)


def _router(x, w_gate):
    col = lambda dt: jax.ShapeDtypeStruct((_N, 1), dt)
    out_shapes = (col(jnp.int32), col(jnp.int32), col(jnp.float32),
                  col(jnp.float32), col(jnp.int32), col(jnp.int32),
                  jax.ShapeDtypeStruct((1, _E), jnp.float32))
    colspec = lambda: pl.BlockSpec((_TB, 1), lambda i: (i, 0))
    return pl.pallas_call(
        _router_body,
        grid=(_NTB,),
        in_specs=[
            pl.BlockSpec((_TB, _D), lambda i: (i, 0)),
            pl.BlockSpec((_D, _E), lambda i: (0, 0)),
        ],
        out_specs=(colspec(), colspec(), colspec(), colspec(),
                   colspec(), colspec(),
                   pl.BlockSpec((1, _E), lambda i: (0, 0))),
        out_shape=out_shapes,
        scratch_shapes=[pltpu.VMEM((1, _E), jnp.float32)],
    )(x, w_gate)


def _offsets_body(cnt_ref, e0_ref, e1_ref, r0_ref, r1_ref,
                  blk_ref, pos0_ref, pos1_ref):
    c = jnp.round(cnt_ref[...]).astype(jnp.int32)
    nb = (c + (_BLK - 1)) >> _BLK_SHIFT
    cpad = (nb << _BLK_SHIFT).astype(jnp.float32)
    f = jax.lax.broadcasted_iota(jnp.int32, (_E, _E), 0)
    e = jax.lax.broadcasted_iota(jnp.int32, (_E, _E), 1)
    ut = (f < e).astype(jnp.float32)
    off = jnp.dot(cpad, ut, preferred_element_type=jnp.float32)
    ib = jax.lax.broadcasted_iota(jnp.int32, (_NB, _E), 0) * _BLK
    le = (off.astype(jnp.int32) <= ib).astype(jnp.int32)
    blk = jnp.sum(le, axis=1, keepdims=True) - 1
    blk_ref[...] = jnp.clip(blk, 0, _E - 1)
    cols = jax.lax.broadcasted_iota(jnp.int32, (_N, _E), 1)
    gath = lambda ev: jnp.sum((cols == ev).astype(jnp.float32) * off,
                              axis=1, keepdims=True)
    pos0_ref[...] = (gath(e0_ref[...]) +
                     r0_ref[...].astype(jnp.float32)).astype(jnp.int32)
    pos1_ref[...] = (gath(e1_ref[...]) +
                     r1_ref[...].astype(jnp.float32)).astype(jnp.int32)


def _offsets(cnt, e0, e1, r0, r1):
    full = lambda: pl.BlockSpec((_N, 1), lambda: (0, 0))
    return pl.pallas_call(
        _offsets_body,
        in_specs=[pl.BlockSpec((1, _E), lambda: (0, 0)),
                  full(), full(), full(), full()],
        out_specs=(pl.BlockSpec((_NB, 1), lambda: (0, 0)),
                   full(), full()),
        out_shape=(jax.ShapeDtypeStruct((_NB, 1), jnp.int32),
                   jax.ShapeDtypeStruct((_N, 1), jnp.int32),
                   jax.ShapeDtypeStruct((_N, 1), jnp.int32)),
    )(cnt, e0, e1, r0, r1)


def _gmm_body(m_ref, xs_ref, gs_ref, w1_ref, w2_ref, ys_ref):
    h = jnp.maximum(
        jnp.dot(xs_ref[...], w1_ref[0], preferred_element_type=jnp.float32),
        0.0)
    o = jnp.dot(h, w2_ref[0], preferred_element_type=jnp.float32)
    ys_ref[...] = o * gs_ref[...]


def _grouped_matmul(xs, gs, W1, W2, blk_expert):
    grid_spec = pltpu.PrefetchScalarGridSpec(
        num_scalar_prefetch=1,
        grid=(_NB,),
        in_specs=[
            pl.BlockSpec((_BLK, _D), lambda i, m: (i, 0)),
            pl.BlockSpec((_BLK, 1), lambda i, m: (i, 0)),
            pl.BlockSpec((1, _D, _H), lambda i, m: (m[i], 0, 0)),
            pl.BlockSpec((1, _H, _D), lambda i, m: (m[i], 0, 0)),
        ],
        out_specs=pl.BlockSpec((_BLK, _D), lambda i, m: (i, 0)),
    )
    return pl.pallas_call(
        _gmm_body,
        grid_spec=grid_spec,
        out_shape=jax.ShapeDtypeStruct((_P, _D), jnp.float32),
    )(blk_expert, xs, gs.reshape(_P, 1), W1, W2)


def _dispatch_body(x_hbm, g0_hbm, g1_hbm, pos0_hbm, pos1_hbm,
                   xs_hbm, gs_hbm,
                   g0_v, g1_v, pos0_v, pos1_v, xrow_v,
                   sem_m, sem_r0, sem_r1, sem_r2, sem_r3,
                   sem_w0, sem_w1, sem_w2, sem_w3):
    wid = lax.axis_index("s") * 2 + lax.axis_index("c")
    base = wid * _CHUNK
    sem_r = (sem_r0, sem_r1, sem_r2, sem_r3)
    sem_w = (sem_w0, sem_w1, sem_w2, sem_w3)

    def fire_read(t):
        return pltpu.async_copy(x_hbm.at[pl.ds(base + t * 16, 16)],
                                xrow_v.at[t % 4], sem_r[t % 4])

    reads = {t: fire_read(t) for t in range(4)}
    meta = [
        pltpu.async_copy(g0_hbm.at[pl.ds(base, _CHUNK)], g0_v, sem_m),
        pltpu.async_copy(g1_hbm.at[pl.ds(base, _CHUNK)], g1_v, sem_m),
        pltpu.async_copy(pos0_hbm.at[pl.ds(base, _CHUNK)], pos0_v, sem_m),
        pltpu.async_copy(pos1_hbm.at[pl.ds(base, _CHUNK)], pos1_v, sem_m),
    ]
    for d in meta:
        d.wait()
    tail = [
        pltpu.async_copy(g0_v, gs_hbm.at[pos0_v], sem_m),
        pltpu.async_copy(g1_v, gs_hbm.at[pos1_v], sem_m),
    ]

    writes = {}
    for t in range(_NG):
        sl = pl.ds(t * 16, 16)
        reads[t].wait()
        writes[t] = (
            pltpu.async_copy(xrow_v.at[t % 4], xs_hbm.at[pos0_v[sl]],
                             sem_w[t % 4]),
            pltpu.async_copy(xrow_v.at[t % 4], xs_hbm.at[pos1_v[sl]],
                             sem_w[t % 4]),
        )
        if t + 4 < _NG:
            writes[t][0].wait()
            writes[t][1].wait()
            reads[t + 4] = fire_read(t + 4)
    for t in range(_NG - 4, _NG):
        writes[t][0].wait()
        writes[t][1].wait()
    for d in tail:
        d.wait()


def _dispatch(x, g0, g1, pos0, pos1):
    mesh = plsc.VectorSubcoreMesh(core_axis_name="c", subcore_axis_name="s")
    f = pl.kernel(
        _dispatch_body,
        mesh=mesh,
        out_type=(jax.ShapeDtypeStruct((_P, _D), jnp.float32),
                  jax.ShapeDtypeStruct((_P,), jnp.float32)),
        scratch_types=[
            pltpu.VMEM((_CHUNK,), jnp.float32),
            pltpu.VMEM((_CHUNK,), jnp.float32),
            pltpu.VMEM((_CHUNK,), jnp.int32),
            pltpu.VMEM((_CHUNK,), jnp.int32),
            pltpu.VMEM((4, 16, _D), jnp.float32),
            pltpu.SemaphoreType.DMA,
            pltpu.SemaphoreType.DMA,
            pltpu.SemaphoreType.DMA,
            pltpu.SemaphoreType.DMA,
            pltpu.SemaphoreType.DMA,
            pltpu.SemaphoreType.DMA,
            pltpu.SemaphoreType.DMA,
            pltpu.SemaphoreType.DMA,
            pltpu.SemaphoreType.DMA,
        ],
        compiler_params=pltpu.CompilerParams(needs_layout_passes=False),
    )
    return f(x, g0, g1, pos0, pos1)


def _combine_body(ys_hbm, pos0_hbm, pos1_hbm, y_hbm,
                  pos0_v, pos1_v, a_v, b_v, o_v,
                  sem_r0, sem_r1, sem_w0, sem_w1):
    wid = lax.axis_index("s") * 2 + lax.axis_index("c")
    base = wid * _CHUNK
    sem_r = (sem_r0, sem_r1)
    sem_w = (sem_w0, sem_w1)
    pltpu.sync_copy(pos0_hbm.at[pl.ds(base, _CHUNK)], pos0_v)
    pltpu.sync_copy(pos1_hbm.at[pl.ds(base, _CHUNK)], pos1_v)

    def fire(t):
        p = t & 1
        sl = pl.ds(t * 16, 16)
        return (pltpu.async_copy(ys_hbm.at[pos0_v[sl]], a_v.at[p], sem_r[p]),
                pltpu.async_copy(ys_hbm.at[pos1_v[sl]], b_v.at[p], sem_r[p]))

    reads = {0: fire(0), 1: fire(1)}
    writes = {}
    for t in range(_NG):
        p = t & 1
        reads[t][0].wait()
        reads[t][1].wait()
        if t >= 2:
            writes[t - 2].wait()

        def row(j, _):
            def col(c, _):
                csl = pl.ds(c * 16, 16)
                o_v[p, j, csl] = a_v[p, j, csl] + b_v[p, j, csl]
                return 0
            lax.fori_loop(0, _D // 16, col, 0)
            return 0
        lax.fori_loop(0, 16, row, 0)
        writes[t] = pltpu.async_copy(
            o_v.at[p], y_hbm.at[pl.ds(base + t * 16, 16)], sem_w[p])
        if t + 2 < _NG:
            reads[t + 2] = fire(t + 2)
    writes[_NG - 2].wait()
    writes[_NG - 1].wait()


def _combine(ys, pos0, pos1):
    mesh = plsc.VectorSubcoreMesh(core_axis_name="c", subcore_axis_name="s")
    f = pl.kernel(
        _combine_body,
        mesh=mesh,
        out_type=jax.ShapeDtypeStruct((_N, _D), jnp.float32),
        scratch_types=[
            pltpu.VMEM((_CHUNK,), jnp.int32),
            pltpu.VMEM((_CHUNK,), jnp.int32),
            pltpu.VMEM((2, 16, _D), jnp.float32),
            pltpu.VMEM((2, 16, _D), jnp.float32),
            pltpu.VMEM((2, 16, _D), jnp.float32),
            pltpu.SemaphoreType.DMA,
            pltpu.SemaphoreType.DMA,
            pltpu.SemaphoreType.DMA,
            pltpu.SemaphoreType.DMA,
        ],
        compiler_params=pltpu.CompilerParams(needs_layout_passes=False),
    )
    return f(ys, pos0, pos1)


def kernel(x, w_gate, W1, W2):
    e0, e1, g0, g1, r0, r1, cnt = _router(x, w_gate)
    blk_expert, pos0, pos1 = _offsets(cnt, e0, e1, r0, r1)
    pos0, pos1 = pos0.reshape(_N), pos1.reshape(_N)
    xs, gs = _dispatch(x, g0.reshape(_N), g1.reshape(_N), pos0, pos1)
    ys = _grouped_matmul(xs, gs, W1, W2, blk_expert.reshape(_NB))
    return _combine(ys, pos0, pos1)


# speedup vs baseline: 1.1414x; 1.0036x over previous
import jax
import jax.numpy as jnp
from jax import lax
from jax.experimental import pallas as pl
from jax.experimental.pallas import tpu as pltpu
from jax.experimental.pallas import tpu_sc as plsc

_N, _D, _H, _E, _TOPK = 4096, 1024, 1024, 8, 2
_NW = 32
_CHUNK = _N // _NW
_NG = _CHUNK // 16
_TB = 512
_NTB = _N // _TB
_BLK = 256
_BLK_SHIFT = 8
_NB = (_N * _TOPK) // _BLK + _E
_P = _NB * _BLK


def _router_body(x_ref, wg_ref, e0_ref, e1_ref, g0_ref, g1_ref,
                 r0_ref, r1_ref, cnt_ref, cnt_acc):
    i = pl.program_id(0)

    @pl.when(i == 0)
    def _():
        cnt_acc[...] = jnp.zeros_like(cnt_acc)

    logits = jnp.dot(x_ref[...], wg_ref[...],
                     preferred_element_type=jnp.float32)
    colsi = jax.lax.broadcasted_iota(jnp.int32, (_TB, _E), 1)
    m0 = jnp.max(logits, axis=1, keepdims=True)
    e0 = jnp.min(jnp.where(logits == m0, colsi, _E), axis=1, keepdims=True)
    oh0 = (colsi == e0).astype(jnp.float32)
    l1 = jnp.where(colsi == e0, -1e30, logits)
    m1 = jnp.max(l1, axis=1, keepdims=True)
    e1 = jnp.min(jnp.where(l1 == m1, colsi, _E), axis=1, keepdims=True)
    oh1 = (colsi == e1).astype(jnp.float32)
    g0 = 1.0 / (1.0 + jnp.exp(m1 - m0))
    g1 = 1.0 - g0
    rows = jax.lax.broadcasted_iota(jnp.int32, (_TB, _TB), 0)
    cols = jax.lax.broadcasted_iota(jnp.int32, (_TB, _TB), 1)
    lt = (cols < rows).astype(jnp.float32)
    cum0 = jnp.dot(lt, oh0, preferred_element_type=jnp.float32)
    cum1 = jnp.dot(lt, oh1, preferred_element_type=jnp.float32)
    cnt = cnt_acc[...]
    tot0 = jnp.sum(oh0, axis=0, keepdims=True)
    tot1 = jnp.sum(oh1, axis=0, keepdims=True)
    r0 = jnp.sum(oh0 * (cnt + cum0), axis=1, keepdims=True)
    r1 = jnp.sum(oh1 * (cnt + tot0 + cum1), axis=1, keepdims=True)
    new_cnt = cnt + tot0 + tot1
    cnt_acc[...] = new_cnt
    cnt_ref[...] = new_cnt
    e0_ref[...] = e0
    e1_ref[...] = e1
    g0_ref[...] = g0
    g1_ref[...] = g1
    r0_ref[...] = r0.astype(jnp.int32)
    r1_ref[...] = r1.astype(jnp.int32)


def _router(x, w_gate):
    col = lambda dt: jax.ShapeDtypeStruct((_N, 1), dt)
    out_shapes = (col(jnp.int32), col(jnp.int32), col(jnp.float32),
                  col(jnp.float32), col(jnp.int32), col(jnp.int32),
                  jax.ShapeDtypeStruct((1, _E), jnp.float32))
    colspec = lambda: pl.BlockSpec((_TB, 1), lambda i: (i, 0))
    return pl.pallas_call(
        _router_body,
        grid=(_NTB,),
        in_specs=[
            pl.BlockSpec((_TB, _D), lambda i: (i, 0)),
            pl.BlockSpec((_D, _E), lambda i: (0, 0)),
        ],
        out_specs=(colspec(), colspec(), colspec(), colspec(),
                   colspec(), colspec(),
                   pl.BlockSpec((1, _E), lambda i: (0, 0))),
        out_shape=out_shapes,
        scratch_shapes=[pltpu.VMEM((1, _E), jnp.float32)],
    )(x, w_gate)


def _offsets_body(cnt_ref, e0_ref, e1_ref, r0_ref, r1_ref,
                  blk_ref, pos0_ref, pos1_ref):
    c = jnp.round(cnt_ref[...]).astype(jnp.int32)
    nb = (c + (_BLK - 1)) >> _BLK_SHIFT
    cpad = (nb << _BLK_SHIFT).astype(jnp.float32)
    f = jax.lax.broadcasted_iota(jnp.int32, (_E, _E), 0)
    e = jax.lax.broadcasted_iota(jnp.int32, (_E, _E), 1)
    ut = (f < e).astype(jnp.float32)
    off = jnp.dot(cpad, ut, preferred_element_type=jnp.float32)
    ib = jax.lax.broadcasted_iota(jnp.int32, (_NB, _E), 0) * _BLK
    le = (off.astype(jnp.int32) <= ib).astype(jnp.int32)
    blk = jnp.sum(le, axis=1, keepdims=True) - 1
    blk_ref[...] = jnp.clip(blk, 0, _E - 1)
    cols = jax.lax.broadcasted_iota(jnp.int32, (_N, _E), 1)
    gath = lambda ev: jnp.sum((cols == ev).astype(jnp.float32) * off,
                              axis=1, keepdims=True)
    pos0_ref[...] = (gath(e0_ref[...]) +
                     r0_ref[...].astype(jnp.float32)).astype(jnp.int32)
    pos1_ref[...] = (gath(e1_ref[...]) +
                     r1_ref[...].astype(jnp.float32)).astype(jnp.int32)


def _offsets(cnt, e0, e1, r0, r1):
    full = lambda: pl.BlockSpec((_N, 1), lambda: (0, 0))
    return pl.pallas_call(
        _offsets_body,
        in_specs=[pl.BlockSpec((1, _E), lambda: (0, 0)),
                  full(), full(), full(), full()],
        out_specs=(pl.BlockSpec((_NB, 1), lambda: (0, 0)),
                   full(), full()),
        out_shape=(jax.ShapeDtypeStruct((_NB, 1), jnp.int32),
                   jax.ShapeDtypeStruct((_N, 1), jnp.int32),
                   jax.ShapeDtypeStruct((_N, 1), jnp.int32)),
    )(cnt, e0, e1, r0, r1)


def _gmm_body(m_ref, xs_ref, gs_ref, w1_ref, w2_ref, ys_ref):
    h = jnp.maximum(
        jnp.dot(xs_ref[...], w1_ref[0], preferred_element_type=jnp.float32),
        0.0)
    o = jnp.dot(h, w2_ref[0], preferred_element_type=jnp.float32)
    ys_ref[...] = o * gs_ref[...]


def _grouped_matmul(xs, gs, W1, W2, blk_expert):
    grid_spec = pltpu.PrefetchScalarGridSpec(
        num_scalar_prefetch=1,
        grid=(_NB,),
        in_specs=[
            pl.BlockSpec((_BLK, _D), lambda i, m: (i, 0)),
            pl.BlockSpec((_BLK, 1), lambda i, m: (i, 0)),
            pl.BlockSpec((1, _D, _H), lambda i, m: (m[i], 0, 0)),
            pl.BlockSpec((1, _H, _D), lambda i, m: (m[i], 0, 0)),
        ],
        out_specs=pl.BlockSpec((_BLK, _D), lambda i, m: (i, 0)),
    )
    return pl.pallas_call(
        _gmm_body,
        grid_spec=grid_spec,
        out_shape=jax.ShapeDtypeStruct((_P, _D), jnp.float32),
    )(blk_expert, xs, gs.reshape(_P, 1), W1, W2)


def _dispatch_body(x_hbm, g0_hbm, g1_hbm, pos0_hbm, pos1_hbm,
                   xs_hbm, gs_hbm,
                   g0_v, g1_v, pos0_v, pos1_v, xrow_v,
                   sem_m, sem_r0, sem_r1, sem_r2, sem_r3,
                   sem_w0, sem_w1, sem_w2, sem_w3):
    wid = lax.axis_index("s") * 2 + lax.axis_index("c")
    base = wid * _CHUNK
    sem_r = (sem_r0, sem_r1, sem_r2, sem_r3)
    sem_w = (sem_w0, sem_w1, sem_w2, sem_w3)

    def fire_read(t):
        return pltpu.async_copy(x_hbm.at[pl.ds(base + t * 16, 16)],
                                xrow_v.at[t % 4], sem_r[t % 4])

    reads = {t: fire_read(t) for t in range(4)}
    meta = [
        pltpu.async_copy(g0_hbm.at[pl.ds(base, _CHUNK)], g0_v, sem_m),
        pltpu.async_copy(g1_hbm.at[pl.ds(base, _CHUNK)], g1_v, sem_m),
        pltpu.async_copy(pos0_hbm.at[pl.ds(base, _CHUNK)], pos0_v, sem_m),
        pltpu.async_copy(pos1_hbm.at[pl.ds(base, _CHUNK)], pos1_v, sem_m),
    ]
    for d in meta:
        d.wait()
    tail = [
        pltpu.async_copy(g0_v, gs_hbm.at[pos0_v], sem_m),
        pltpu.async_copy(g1_v, gs_hbm.at[pos1_v], sem_m),
    ]

    writes = {}
    for t in range(_NG):
        sl = pl.ds(t * 16, 16)
        reads[t].wait()
        writes[t] = (
            pltpu.async_copy(xrow_v.at[t % 4], xs_hbm.at[pos0_v[sl]],
                             sem_w[t % 4]),
            pltpu.async_copy(xrow_v.at[t % 4], xs_hbm.at[pos1_v[sl]],
                             sem_w[t % 4]),
        )
        if t + 4 < _NG:
            writes[t][0].wait()
            writes[t][1].wait()
            reads[t + 4] = fire_read(t + 4)
    for t in range(_NG - 4, _NG):
        writes[t][0].wait()
        writes[t][1].wait()
    for d in tail:
        d.wait()


def _dispatch(x, g0, g1, pos0, pos1):
    mesh = plsc.VectorSubcoreMesh(core_axis_name="c", subcore_axis_name="s")
    f = pl.kernel(
        _dispatch_body,
        mesh=mesh,
        out_type=(jax.ShapeDtypeStruct((_P, _D), jnp.float32),
                  jax.ShapeDtypeStruct((_P,), jnp.float32)),
        scratch_types=[
            pltpu.VMEM((_CHUNK,), jnp.float32),
            pltpu.VMEM((_CHUNK,), jnp.float32),
            pltpu.VMEM((_CHUNK,), jnp.int32),
            pltpu.VMEM((_CHUNK,), jnp.int32),
            pltpu.VMEM((4, 16, _D), jnp.float32),
            pltpu.SemaphoreType.DMA,
            pltpu.SemaphoreType.DMA,
            pltpu.SemaphoreType.DMA,
            pltpu.SemaphoreType.DMA,
            pltpu.SemaphoreType.DMA,
            pltpu.SemaphoreType.DMA,
            pltpu.SemaphoreType.DMA,
            pltpu.SemaphoreType.DMA,
            pltpu.SemaphoreType.DMA,
        ],
        compiler_params=pltpu.CompilerParams(needs_layout_passes=False),
    )
    return f(x, g0, g1, pos0, pos1)


def _combine_body(ys_hbm, pos0_hbm, pos1_hbm, y_hbm,
                  pos0_v, pos1_v, a_v, b_v, o_v,
                  sem_r0, sem_r1, sem_w0, sem_w1):
    wid = lax.axis_index("s") * 2 + lax.axis_index("c")
    base = wid * _CHUNK
    sem_r = (sem_r0, sem_r1)
    sem_w = (sem_w0, sem_w1)
    pltpu.sync_copy(pos0_hbm.at[pl.ds(base, _CHUNK)], pos0_v)
    pltpu.sync_copy(pos1_hbm.at[pl.ds(base, _CHUNK)], pos1_v)

    def fire(t):
        p = t & 1
        sl = pl.ds(t * 16, 16)
        return (pltpu.async_copy(ys_hbm.at[pos0_v[sl]], a_v.at[p], sem_r[p]),
                pltpu.async_copy(ys_hbm.at[pos1_v[sl]], b_v.at[p], sem_r[p]))

    reads = {0: fire(0), 1: fire(1)}
    writes = {}
    for t in range(_NG):
        p = t & 1
        reads[t][0].wait()
        reads[t][1].wait()
        if t >= 2:
            writes[t - 2].wait()

        def row(j, _):
            def col(c, _):
                csl = pl.ds(c * 16, 16)
                o_v[p, j, csl] = a_v[p, j, csl] + b_v[p, j, csl]
                return 0
            lax.fori_loop(0, _D // 16, col, 0)
            return 0
        lax.fori_loop(0, 16, row, 0)
        writes[t] = pltpu.async_copy(
            o_v.at[p], y_hbm.at[pl.ds(base + t * 16, 16)], sem_w[p])
        if t + 2 < _NG:
            reads[t + 2] = fire(t + 2)
    writes[_NG - 2].wait()
    writes[_NG - 1].wait()


def _combine(ys, pos0, pos1):
    mesh = plsc.VectorSubcoreMesh(core_axis_name="c", subcore_axis_name="s")
    f = pl.kernel(
        _combine_body,
        mesh=mesh,
        out_type=jax.ShapeDtypeStruct((_N, _D), jnp.float32),
        scratch_types=[
            pltpu.VMEM((_CHUNK,), jnp.int32),
            pltpu.VMEM((_CHUNK,), jnp.int32),
            pltpu.VMEM((2, 16, _D), jnp.float32),
            pltpu.VMEM((2, 16, _D), jnp.float32),
            pltpu.VMEM((2, 16, _D), jnp.float32),
            pltpu.SemaphoreType.DMA,
            pltpu.SemaphoreType.DMA,
            pltpu.SemaphoreType.DMA,
            pltpu.SemaphoreType.DMA,
        ],
        compiler_params=pltpu.CompilerParams(needs_layout_passes=False),
    )
    return f(ys, pos0, pos1)


def kernel(x, w_gate, W1, W2):
    e0, e1, g0, g1, r0, r1, cnt = _router(x, w_gate)
    blk_expert, pos0, pos1 = _offsets(cnt, e0, e1, r0, r1)
    pos0, pos1 = pos0.reshape(_N), pos1.reshape(_N)
    xs, gs = _dispatch(x, g0.reshape(_N), g1.reshape(_N), pos0, pos1)
    ys = _grouped_matmul(xs, gs, W1, W2, blk_expert.reshape(_NB))
    return _combine(ys, pos0, pos1)
